# Initial kernel scaffold; baseline (speedup 1.0000x reference)
import jax
import jax.numpy as jnp
from jax.experimental import pallas as pl


def kernel(x):
    raise NotImplementedError("write your pallas kernel here")


# speedup vs baseline: 2.6052x; 2.6052x over previous
import functools

import numpy as np
import jax
import jax.numpy as jnp
from jax.experimental import pallas as pl
from jax.experimental.pallas import tpu as pltpu


def _bilinear_matrix(out_size, in_size):
    if in_size == 1:
        return jnp.ones((out_size, 1), jnp.float32)
    scale = np.float32((in_size - 1) / (out_size - 1))
    pos = np.arange(out_size, dtype=np.float32) * scale
    low = np.clip(np.floor(pos).astype(np.int64), 0, in_size - 2)
    frac = pos - low.astype(np.float32)
    m = np.zeros((out_size, in_size), np.float32)
    rows = np.arange(out_size)
    m[rows, low] += 1.0 - frac
    m[rows, low + 1] += frac
    return jnp.asarray(m)


def _upsample_once_kernel(n_rep, ah_ref, x_ref, awt_ref, o_ref):
    x = x_ref[0, 0]
    tmp = jnp.dot(x, awt_ref[...], preferred_element_type=jnp.float32)
    b = tmp.shape[0] // awt_ref.shape[0]
    for i in range(b):
        rows = tmp[i * awt_ref.shape[0]:(i + 1) * awt_ref.shape[0]]
        out = jnp.dot(ah_ref[...], rows,
                      preferred_element_type=jnp.float32)
        for c in range(n_rep):
            o_ref[i, c] = out


def kernel(x, out_hw=(224, 224)):
    assert x.ndim == 4, "expected NCHW input"
    n, c_in, h, w = x.shape
    oh, ow = out_hw
    c_out = c_in if c_in >= 3 else 3

    a_h = _bilinear_matrix(oh, h)
    a_wt = _bilinear_matrix(ow, w).T

    if c_in == 1:
        img_block = 4 if n % 4 == 0 else 1
        x2 = x.reshape(n, 1, h, w)
        grid = (n // img_block,)
        call = pl.pallas_call(
            functools.partial(_upsample_once_kernel, c_out),
            out_shape=jax.ShapeDtypeStruct((n, c_out, oh, ow), x.dtype),
            grid_spec=pltpu.PrefetchScalarGridSpec(
                num_scalar_prefetch=0,
                grid=grid,
                in_specs=[
                    pl.BlockSpec((oh, h), lambda i: (0, 0)),
                    pl.BlockSpec((1, 1, img_block * h, w),
                                 lambda i: (i, 0, 0, 0)),
                    pl.BlockSpec((w, ow), lambda i: (0, 0)),
                ],
                out_specs=pl.BlockSpec((img_block, c_out, oh, ow),
                                       lambda i: (i, 0, 0, 0)),
            ),
            compiler_params=pltpu.CompilerParams(
                dimension_semantics=("parallel",)),
            cost_estimate=pl.CostEstimate(
                flops=2 * n * (oh * h * w + oh * ow * w),
                transcendentals=0,
                bytes_accessed=(n * h * w + n * c_out * oh * ow)
                * x.dtype.itemsize,
            ),
        )
        return call(a_h, x2.reshape(n // img_block, 1, img_block * h, w), a_wt)

    def _general_kernel(ah_ref, x_ref, awt_ref, o_ref):
        xi = x_ref[0, 0]
        tmp = jnp.dot(ah_ref[...], xi, preferred_element_type=jnp.float32)
        o_ref[0, 0] = jnp.dot(tmp, awt_ref[...],
                              preferred_element_type=jnp.float32)

    x_map = (lambda nn_, cc: (nn_, cc, 0, 0)) if c_in >= 3 else (
        lambda nn_, cc: (nn_, cc % c_in, 0, 0))
    return pl.pallas_call(
        _general_kernel,
        out_shape=jax.ShapeDtypeStruct((n, c_out, oh, ow), x.dtype),
        grid_spec=pltpu.PrefetchScalarGridSpec(
            num_scalar_prefetch=0,
            grid=(n, c_out),
            in_specs=[
                pl.BlockSpec((oh, h), lambda nn_, cc: (0, 0)),
                pl.BlockSpec((1, 1, h, w), x_map),
                pl.BlockSpec((w, ow), lambda nn_, cc: (0, 0)),
            ],
            out_specs=pl.BlockSpec((1, 1, oh, ow),
                                   lambda nn_, cc: (nn_, cc, 0, 0)),
        ),
        compiler_params=pltpu.CompilerParams(
            dimension_semantics=("parallel", "parallel")),
    )(a_h, x, a_wt)


# speedup vs baseline: 2.7175x; 1.0431x over previous
import functools

import numpy as np
import jax
import jax.numpy as jnp
from jax.experimental import pallas as pl
from jax.experimental.pallas import tpu as pltpu


def _bilinear_matrix(out_size, in_size):
    if in_size == 1:
        return jnp.ones((out_size, 1), jnp.float32)
    scale = np.float32((in_size - 1) / (out_size - 1))
    pos = np.arange(out_size, dtype=np.float32) * scale
    low = np.clip(np.floor(pos).astype(np.int64), 0, in_size - 2)
    frac = pos - low.astype(np.float32)
    m = np.zeros((out_size, in_size), np.float32)
    rows = np.arange(out_size)
    m[rows, low] += 1.0 - frac
    m[rows, low + 1] += frac
    return jnp.asarray(m)


def _upsample_once_kernel(n_rep, ah_ref, x_ref, awt_ref, o_ref):
    b, _, h, w = x_ref.shape
    x = x_ref[...].reshape(b * h, w)
    tmp = jnp.dot(x, awt_ref[...], preferred_element_type=jnp.float32)
    for i in range(b):
        rows = tmp[i * h:(i + 1) * h]
        out = jnp.dot(ah_ref[...], rows,
                      preferred_element_type=jnp.float32)
        for c in range(n_rep):
            o_ref[i, c] = out


def kernel(x, out_hw=(224, 224)):
    assert x.ndim == 4, "expected NCHW input"
    n, c_in, h, w = x.shape
    oh, ow = out_hw
    c_out = c_in if c_in >= 3 else 3

    a_h = _bilinear_matrix(oh, h)
    a_wt = _bilinear_matrix(ow, w).T

    if c_in == 1:
        img_block = 4 if n % 4 == 0 else 1
        call = pl.pallas_call(
            functools.partial(_upsample_once_kernel, c_out),
            out_shape=jax.ShapeDtypeStruct((n, c_out, oh, ow), x.dtype),
            grid_spec=pltpu.PrefetchScalarGridSpec(
                num_scalar_prefetch=0,
                grid=(n // img_block,),
                in_specs=[
                    pl.BlockSpec((oh, h), lambda i: (0, 0)),
                    pl.BlockSpec((img_block, 1, h, w),
                                 lambda i: (i, 0, 0, 0)),
                    pl.BlockSpec((w, ow), lambda i: (0, 0)),
                ],
                out_specs=pl.BlockSpec((img_block, c_out, oh, ow),
                                       lambda i: (i, 0, 0, 0)),
            ),
            compiler_params=pltpu.CompilerParams(
                dimension_semantics=("parallel",)),
            cost_estimate=pl.CostEstimate(
                flops=2 * n * (oh * h * w + oh * ow * w),
                transcendentals=0,
                bytes_accessed=(n * h * w + n * c_out * oh * ow)
                * x.dtype.itemsize,
            ),
        )
        return call(a_h, x, a_wt)

    def _general_kernel(ah_ref, x_ref, awt_ref, o_ref):
        xi = x_ref[0, 0]
        tmp = jnp.dot(ah_ref[...], xi, preferred_element_type=jnp.float32)
        o_ref[0, 0] = jnp.dot(tmp, awt_ref[...],
                              preferred_element_type=jnp.float32)

    x_map = (lambda nn_, cc: (nn_, cc, 0, 0)) if c_in >= 3 else (
        lambda nn_, cc: (nn_, cc % c_in, 0, 0))
    return pl.pallas_call(
        _general_kernel,
        out_shape=jax.ShapeDtypeStruct((n, c_out, oh, ow), x.dtype),
        grid_spec=pltpu.PrefetchScalarGridSpec(
            num_scalar_prefetch=0,
            grid=(n, c_out),
            in_specs=[
                pl.BlockSpec((oh, h), lambda nn_, cc: (0, 0)),
                pl.BlockSpec((1, 1, h, w), x_map),
                pl.BlockSpec((w, ow), lambda nn_, cc: (0, 0)),
            ],
            out_specs=pl.BlockSpec((1, 1, oh, ow),
                                   lambda nn_, cc: (nn_, cc, 0, 0)),
        ),
        compiler_params=pltpu.CompilerParams(
            dimension_semantics=("parallel", "parallel")),
    )(a_h, x, a_wt)


# speedup vs baseline: 7.0807x; 2.6056x over previous
import functools

import numpy as np
import jax
import jax.numpy as jnp
from jax.experimental import pallas as pl
from jax.experimental.pallas import tpu as pltpu


def _bilinear_matrix(out_size, in_size):
    if in_size == 1:
        return jnp.ones((out_size, 1), jnp.float32)
    scale = np.float32((in_size - 1) / (out_size - 1))
    pos = np.arange(out_size, dtype=np.float32) * scale
    low = np.clip(np.floor(pos).astype(np.int64), 0, in_size - 2)
    frac = pos - low.astype(np.float32)
    m = np.zeros((out_size, in_size), np.float32)
    rows = np.arange(out_size)
    m[rows, low] += 1.0 - frac
    m[rows, low + 1] += frac
    return jnp.asarray(m)


def _rows_kernel(h, oh, n_rep, aw_ref, x_ref, o_ref):
    i = pl.program_id(0)
    pos = i.astype(jnp.float32) * np.float32((h - 1) / (oh - 1))
    low = jnp.minimum(jnp.floor(pos).astype(jnp.int32), h - 2)
    frac = pos - low.astype(jnp.float32)
    x0 = x_ref[low]
    x1 = x_ref[low + 1]
    tmp = (1.0 - frac) * x0 + frac * x1
    out = jnp.dot(aw_ref[...], tmp,
                  preferred_element_type=jnp.float32)
    for c in range(n_rep):
        o_ref[c, 0] = out


def kernel(x, out_hw=(224, 224)):
    assert x.ndim == 4, "expected NCHW input"
    n, c_in, h, w = x.shape
    oh, ow = out_hw
    c_out = c_in if c_in >= 3 else 3

    if c_in == 1 and h > 1:
        a_w = _bilinear_matrix(ow, w)
        xt = jnp.transpose(x, (1, 2, 3, 0)).reshape(h, w, n)
        out_t = pl.pallas_call(
            functools.partial(_rows_kernel, h, oh, c_out),
            out_shape=jax.ShapeDtypeStruct((c_out, oh, ow, n), x.dtype),
            grid_spec=pltpu.PrefetchScalarGridSpec(
                num_scalar_prefetch=0,
                grid=(oh,),
                in_specs=[
                    pl.BlockSpec((ow, w), lambda i: (0, 0)),
                    pl.BlockSpec((h, w, n), lambda i: (0, 0, 0)),
                ],
                out_specs=pl.BlockSpec((c_out, 1, ow, n),
                                       lambda i: (0, i, 0, 0)),
            ),
            compiler_params=pltpu.CompilerParams(
                dimension_semantics=("parallel",)),
            cost_estimate=pl.CostEstimate(
                flops=2 * n * (oh * w + oh * ow * w),
                transcendentals=0,
                bytes_accessed=(n * h * w + n * c_out * oh * ow)
                * x.dtype.itemsize,
            ),
        )(a_w, xt)
        return jnp.transpose(out_t, (3, 0, 1, 2))

    a_h = _bilinear_matrix(oh, h)
    a_wtt = _bilinear_matrix(ow, w).T

    def _general_kernel(ah_ref, x_ref, awt_ref, o_ref):
        xi = x_ref[0, 0]
        tmp = jnp.dot(ah_ref[...], xi, preferred_element_type=jnp.float32)
        o_ref[0, 0] = jnp.dot(tmp, awt_ref[...],
                              preferred_element_type=jnp.float32)

    x_map = (lambda nn_, cc: (nn_, cc, 0, 0)) if c_in >= 3 else (
        lambda nn_, cc: (nn_, cc % c_in, 0, 0))
    return pl.pallas_call(
        _general_kernel,
        out_shape=jax.ShapeDtypeStruct((n, c_out, oh, ow), x.dtype),
        grid_spec=pltpu.PrefetchScalarGridSpec(
            num_scalar_prefetch=0,
            grid=(n, c_out),
            in_specs=[
                pl.BlockSpec((oh, h), lambda nn_, cc: (0, 0)),
                pl.BlockSpec((1, 1, h, w), x_map),
                pl.BlockSpec((w, ow), lambda nn_, cc: (0, 0)),
            ],
            out_specs=pl.BlockSpec((1, 1, oh, ow),
                                   lambda nn_, cc: (nn_, cc, 0, 0)),
        ),
        compiler_params=pltpu.CompilerParams(
            dimension_semantics=("parallel", "parallel")),
    )(a_h, x, a_wtt)


# speedup vs baseline: 12.9594x; 1.8302x over previous
import functools

import numpy as np
import jax
import jax.numpy as jnp
from jax.experimental import pallas as pl
from jax.experimental.pallas import tpu as pltpu


def _bilinear_matrix(out_size, in_size):
    if in_size == 1:
        return jnp.ones((out_size, 1), jnp.float32)
    scale = np.float32((in_size - 1) / (out_size - 1))
    pos = np.arange(out_size, dtype=np.float32) * scale
    low = np.clip(np.floor(pos).astype(np.int64), 0, in_size - 2)
    frac = pos - low.astype(np.float32)
    m = np.zeros((out_size, in_size), np.float32)
    rows = np.arange(out_size)
    m[rows, low] += 1.0 - frac
    m[rows, low + 1] += frac
    return jnp.asarray(m)


def _rows_kernel(h, oh, n_rep, tile_oh, aw_ref, x_ref, o_ref):
    base = pl.program_id(0) * tile_oh
    for j in range(tile_oh):
        i = base + j
        pos = i.astype(jnp.float32) * np.float32((h - 1) / (oh - 1))
        low = jnp.minimum(jnp.floor(pos).astype(jnp.int32), h - 2)
        frac = pos - low.astype(jnp.float32)
        x0 = x_ref[low]
        x1 = x_ref[low + 1]
        tmp = (1.0 - frac) * x0 + frac * x1
        out = jnp.dot(aw_ref[...], tmp,
                      preferred_element_type=jnp.float32)
        for c in range(n_rep):
            o_ref[c, j] = out


def kernel(x, out_hw=(224, 224)):
    assert x.ndim == 4, "expected NCHW input"
    n, c_in, h, w = x.shape
    oh, ow = out_hw
    c_out = c_in if c_in >= 3 else 3

    if c_in == 1 and h > 1:
        a_w = _bilinear_matrix(ow, w)
        xt = jnp.transpose(x, (1, 2, 3, 0)).reshape(h, w, n)
        tile_oh = 4 if oh % 4 == 0 else 1
        out_t = pl.pallas_call(
            functools.partial(_rows_kernel, h, oh, c_out, tile_oh),
            out_shape=jax.ShapeDtypeStruct((c_out, oh, ow, n), x.dtype),
            grid_spec=pltpu.PrefetchScalarGridSpec(
                num_scalar_prefetch=0,
                grid=(oh // tile_oh,),
                in_specs=[
                    pl.BlockSpec((ow, w), lambda i: (0, 0)),
                    pl.BlockSpec((h, w, n), lambda i: (0, 0, 0)),
                ],
                out_specs=pl.BlockSpec((c_out, tile_oh, ow, n),
                                       lambda i: (0, i, 0, 0)),
            ),
            compiler_params=pltpu.CompilerParams(
                dimension_semantics=("parallel",)),
            cost_estimate=pl.CostEstimate(
                flops=2 * n * (oh * w + oh * ow * w),
                transcendentals=0,
                bytes_accessed=(n * h * w + n * c_out * oh * ow)
                * x.dtype.itemsize,
            ),
        )(a_w, xt)
        return jnp.transpose(out_t, (3, 0, 1, 2))

    a_h = _bilinear_matrix(oh, h)
    a_wtt = _bilinear_matrix(ow, w).T

    def _general_kernel(ah_ref, x_ref, awt_ref, o_ref):
        xi = x_ref[0, 0]
        tmp = jnp.dot(ah_ref[...], xi, preferred_element_type=jnp.float32)
        o_ref[0, 0] = jnp.dot(tmp, awt_ref[...],
                              preferred_element_type=jnp.float32)

    x_map = (lambda nn_, cc: (nn_, cc, 0, 0)) if c_in >= 3 else (
        lambda nn_, cc: (nn_, cc % c_in, 0, 0))
    return pl.pallas_call(
        _general_kernel,
        out_shape=jax.ShapeDtypeStruct((n, c_out, oh, ow), x.dtype),
        grid_spec=pltpu.PrefetchScalarGridSpec(
            num_scalar_prefetch=0,
            grid=(n, c_out),
            in_specs=[
                pl.BlockSpec((oh, h), lambda nn_, cc: (0, 0)),
                pl.BlockSpec((1, 1, h, w), x_map),
                pl.BlockSpec((w, ow), lambda nn_, cc: (0, 0)),
            ],
            out_specs=pl.BlockSpec((1, 1, oh, ow),
                                   lambda nn_, cc: (nn_, cc, 0, 0)),
        ),
        compiler_params=pltpu.CompilerParams(
            dimension_semantics=("parallel", "parallel")),
    )(a_h, x, a_wtt)


# speedup vs baseline: 12.9747x; 1.0012x over previous
import functools

import numpy as np
import jax
import jax.numpy as jnp
from jax.experimental import pallas as pl
from jax.experimental.pallas import tpu as pltpu


def _bilinear_matrix(out_size, in_size):
    if in_size == 1:
        return jnp.ones((out_size, 1), jnp.float32)
    scale = np.float32((in_size - 1) / (out_size - 1))
    pos = np.arange(out_size, dtype=np.float32) * scale
    low = np.clip(np.floor(pos).astype(np.int64), 0, in_size - 2)
    frac = pos - low.astype(np.float32)
    m = np.zeros((out_size, in_size), np.float32)
    rows = np.arange(out_size)
    m[rows, low] += 1.0 - frac
    m[rows, low + 1] += frac
    return jnp.asarray(m)


def _rows_kernel(h, oh, n_rep, tile_oh, aw_ref, x_ref, o_ref):
    base = pl.program_id(0) * tile_oh
    for j in range(tile_oh):
        i = base + j
        pos = i.astype(jnp.float32) * np.float32((h - 1) / (oh - 1))
        low = jnp.minimum(jnp.floor(pos).astype(jnp.int32), h - 2)
        frac = pos - low.astype(jnp.float32)
        x0 = x_ref[low]
        x1 = x_ref[low + 1]
        tmp = (1.0 - frac) * x0 + frac * x1
        out = jnp.dot(aw_ref[...], tmp,
                      preferred_element_type=jnp.float32)
        for c in range(n_rep):
            o_ref[c, j] = out


def kernel(x, out_hw=(224, 224)):
    assert x.ndim == 4, "expected NCHW input"
    n, c_in, h, w = x.shape
    oh, ow = out_hw
    c_out = c_in if c_in >= 3 else 3

    if c_in == 1 and h > 1:
        a_w = _bilinear_matrix(ow, w)
        xt = jnp.transpose(x, (1, 2, 3, 0)).reshape(h, w, n)
        tile_oh = 8 if oh % 8 == 0 else 1
        out_t = pl.pallas_call(
            functools.partial(_rows_kernel, h, oh, c_out, tile_oh),
            out_shape=jax.ShapeDtypeStruct((c_out, oh, ow, n), x.dtype),
            grid_spec=pltpu.PrefetchScalarGridSpec(
                num_scalar_prefetch=0,
                grid=(oh // tile_oh,),
                in_specs=[
                    pl.BlockSpec((ow, w), lambda i: (0, 0)),
                    pl.BlockSpec((h, w, n), lambda i: (0, 0, 0)),
                ],
                out_specs=pl.BlockSpec((c_out, tile_oh, ow, n),
                                       lambda i: (0, i, 0, 0)),
            ),
            compiler_params=pltpu.CompilerParams(
                dimension_semantics=("parallel",)),
            cost_estimate=pl.CostEstimate(
                flops=2 * n * (oh * w + oh * ow * w),
                transcendentals=0,
                bytes_accessed=(n * h * w + n * c_out * oh * ow)
                * x.dtype.itemsize,
            ),
        )(a_w, xt)
        return jnp.transpose(out_t, (3, 0, 1, 2))

    a_h = _bilinear_matrix(oh, h)
    a_wtt = _bilinear_matrix(ow, w).T

    def _general_kernel(ah_ref, x_ref, awt_ref, o_ref):
        xi = x_ref[0, 0]
        tmp = jnp.dot(ah_ref[...], xi, preferred_element_type=jnp.float32)
        o_ref[0, 0] = jnp.dot(tmp, awt_ref[...],
                              preferred_element_type=jnp.float32)

    x_map = (lambda nn_, cc: (nn_, cc, 0, 0)) if c_in >= 3 else (
        lambda nn_, cc: (nn_, cc % c_in, 0, 0))
    return pl.pallas_call(
        _general_kernel,
        out_shape=jax.ShapeDtypeStruct((n, c_out, oh, ow), x.dtype),
        grid_spec=pltpu.PrefetchScalarGridSpec(
            num_scalar_prefetch=0,
            grid=(n, c_out),
            in_specs=[
                pl.BlockSpec((oh, h), lambda nn_, cc: (0, 0)),
                pl.BlockSpec((1, 1, h, w), x_map),
                pl.BlockSpec((w, ow), lambda nn_, cc: (0, 0)),
            ],
            out_specs=pl.BlockSpec((1, 1, oh, ow),
                                   lambda nn_, cc: (nn_, cc, 0, 0)),
        ),
        compiler_params=pltpu.CompilerParams(
            dimension_semantics=("parallel", "parallel")),
    )(a_h, x, a_wtt)
